# Initial kernel scaffold; baseline (speedup 1.0000x reference)
import jax
import jax.numpy as jnp
from jax.experimental import pallas as pl


def kernel(inputs, token_table, position_table):
    raise NotImplementedError("write your pallas kernel here")


# speedup vs baseline: 4.6286x; 4.6286x over previous
import functools

import jax
import jax.numpy as jnp
from jax import lax
from jax.experimental import pallas as pl
from jax.experimental.pallas import tpu as pltpu
from jax.experimental.pallas import tpu_sc as plsc

NUM_CORES = 2
NUM_SUBCORES = 16
NUM_WORKERS = NUM_CORES * NUM_SUBCORES

CHUNK_BATCH_ROWS = 8


@functools.partial(jax.jit, static_argnums=(3, 4))
def _pos_embed_sc(flat_idx, token_table, pos_block, seq_len, dim):
    n = flat_idx.shape[0]
    per_w = n // NUM_WORKERS
    chunk = CHUNK_BATCH_ROWS * seq_len
    n_chunks = per_w // chunk

    mesh = plsc.VectorSubcoreMesh(
        core_axis_name="c", subcore_axis_name="s",
        num_cores=NUM_CORES, num_subcores=NUM_SUBCORES)

    @functools.partial(
        pl.kernel,
        mesh=mesh,
        compiler_params=pltpu.CompilerParams(use_tc_tiling_on_sc=False),
        out_type=jax.ShapeDtypeStruct((n, dim), jnp.float32),
        scratch_types=[
            pltpu.VMEM((chunk,), jnp.int32),
            pltpu.VMEM((chunk, dim), jnp.float32),
            pltpu.VMEM((chunk, dim), jnp.float32),
            pltpu.SemaphoreType.DMA,
        ],
    )
    def sc_kernel(idx_hbm, tok_hbm, pos_hbm, out_hbm, idx_v, rows_v, pos_v, sem):
        wid = lax.axis_index("s") * NUM_CORES + lax.axis_index("c")
        base = wid * per_w
        pltpu.sync_copy(pos_hbm, pos_v)

        def chunk_body(c, _):
            off = base + c * chunk
            pltpu.sync_copy(idx_hbm.at[pl.ds(off, chunk)], idx_v)
            pltpu.async_copy(tok_hbm.at[idx_v], rows_v, sem).wait()

            def add_body(i):
                rows_v[i, pl.ds(0, 16)] = rows_v[i, pl.ds(0, 16)] + pos_v[i, pl.ds(0, 16)]
                rows_v[i, pl.ds(16, 16)] = rows_v[i, pl.ds(16, 16)] + pos_v[i, pl.ds(16, 16)]

            plsc.parallel_loop(0, chunk, 1, unroll=8)(add_body)
            pltpu.sync_copy(rows_v, out_hbm.at[pl.ds(off, chunk)])
            return ()

        lax.fori_loop(0, n_chunks, chunk_body, ())

    return sc_kernel(flat_idx, token_table, pos_block)


def kernel(inputs, token_table, position_table):
    b, s = inputs.shape
    dim = token_table.shape[1]
    flat_idx = inputs.reshape(b * s).astype(jnp.int32)
    pos_block = jnp.tile(position_table, (CHUNK_BATCH_ROWS, 1))
    out = _pos_embed_sc(flat_idx, token_table, pos_block, s, dim)
    return out.reshape(b, s, dim)


# speedup vs baseline: 5.1267x; 1.1076x over previous
import functools

import jax
import jax.numpy as jnp
from jax import lax
from jax.experimental import pallas as pl
from jax.experimental.pallas import tpu as pltpu
from jax.experimental.pallas import tpu_sc as plsc

NUM_CORES = 2
NUM_SUBCORES = 16
NUM_WORKERS = NUM_CORES * NUM_SUBCORES

CHUNK_BATCH_ROWS = 4
NBUF = 4


@functools.partial(jax.jit, static_argnums=(3, 4))
def _pos_embed_sc(flat_idx, token_table, position_table, seq_len, dim):
    n = flat_idx.shape[0]
    per_w = n // NUM_WORKERS
    chunk = CHUNK_BATCH_ROWS * seq_len
    n_chunks = per_w // chunk
    assert n_chunks % NBUF == 0

    mesh = plsc.VectorSubcoreMesh(
        core_axis_name="c", subcore_axis_name="s",
        num_cores=NUM_CORES, num_subcores=NUM_SUBCORES)

    @functools.partial(
        pl.kernel,
        mesh=mesh,
        compiler_params=pltpu.CompilerParams(use_tc_tiling_on_sc=False),
        out_type=jax.ShapeDtypeStruct((n, dim), jnp.float32),
        scratch_types=(
            [pltpu.VMEM((chunk,), jnp.int32) for _ in range(NBUF)]
            + [pltpu.VMEM((chunk, dim), jnp.float32) for _ in range(NBUF)]
            + [pltpu.VMEM((seq_len, dim), jnp.float32)]
            + [pltpu.SemaphoreType.DMA for _ in range(2 * NBUF)]
        ),
    )
    def sc_kernel(idx_hbm, tok_hbm, pos_hbm, out_hbm, *scratch):
        idx_v = scratch[:NBUF]
        rows_v = scratch[NBUF:2 * NBUF]
        pos_v = scratch[2 * NBUF]
        sem_g = scratch[2 * NBUF + 1:2 * NBUF + 1 + NBUF]
        sem_s = scratch[2 * NBUF + 1 + NBUF:]

        wid = lax.axis_index("s") * NUM_CORES + lax.axis_index("c")
        base = wid * per_w
        pltpu.sync_copy(pos_hbm, pos_v)

        def issue_gather(c, b):
            off = base + c * chunk
            pltpu.sync_copy(idx_hbm.at[pl.ds(off, chunk)], idx_v[b])
            pltpu.async_copy(tok_hbm.at[idx_v[b]], rows_v[b], sem_g[b])

        def wait_gather(b):
            pltpu.make_async_copy(tok_hbm.at[idx_v[b]], rows_v[b], sem_g[b]).wait()

        def wait_scatter(b):
            pltpu.make_async_copy(
                rows_v[b], out_hbm.at[pl.ds(0, chunk)], sem_s[b]).wait()

        def add_positions(rref):
            def s_body(s):
                p0 = pos_v[s, pl.ds(0, 16)]
                p1 = pos_v[s, pl.ds(16, 16)]
                for r in range(CHUNK_BATCH_ROWS):
                    j = r * seq_len + s
                    rref[j, pl.ds(0, 16)] = rref[j, pl.ds(0, 16)] + p0
                    rref[j, pl.ds(16, 16)] = rref[j, pl.ds(16, 16)] + p1

            plsc.parallel_loop(0, seq_len, 1, unroll=2)(s_body)

        for k in range(NBUF - 1):
            issue_gather(k, k)

        def outer(i, _):
            for b in range(NBUF):
                c = i * NBUF + b
                g = c + NBUF - 1
                gb = (b + NBUF - 1) % NBUF

                @pl.when(g < n_chunks)
                def _issue():
                    @pl.when(c >= 1)
                    def _drain():
                        wait_scatter(gb)

                    issue_gather(g, gb)

                wait_gather(b)
                add_positions(rows_v[b])
                off = base + c * chunk
                pltpu.async_copy(rows_v[b], out_hbm.at[pl.ds(off, chunk)], sem_s[b])
            return ()

        lax.fori_loop(0, n_chunks // NBUF, outer, ())

        for b in range(NBUF):
            wait_scatter(b)

    return sc_kernel(flat_idx, token_table, position_table)


def kernel(inputs, token_table, position_table):
    b, s = inputs.shape
    dim = token_table.shape[1]
    flat_idx = inputs.reshape(b * s).astype(jnp.int32)
    out = _pos_embed_sc(flat_idx, token_table, position_table, s, dim)
    return out.reshape(b, s, dim)


# speedup vs baseline: 5.1274x; 1.0001x over previous
import functools

import jax
import jax.numpy as jnp
from jax import lax
from jax.experimental import pallas as pl
from jax.experimental.pallas import tpu as pltpu
from jax.experimental.pallas import tpu_sc as plsc

NUM_CORES = 2
NUM_SUBCORES = 16
NUM_WORKERS = NUM_CORES * NUM_SUBCORES

CHUNK_BATCH_ROWS = 4
NBUF = 4


@functools.partial(jax.jit, static_argnums=(3, 4, 5))
def _pos_embed_sc(flat_idx, token_table, position_table, batch, seq_len, dim):
    n = flat_idx.shape[0]
    rows_per_w = batch // NUM_WORKERS
    per_w = n // NUM_WORKERS
    chunk = CHUNK_BATCH_ROWS * seq_len
    n_chunks = rows_per_w // CHUNK_BATCH_ROWS
    assert n_chunks % NBUF == 0

    mesh = plsc.VectorSubcoreMesh(
        core_axis_name="c", subcore_axis_name="s",
        num_cores=NUM_CORES, num_subcores=NUM_SUBCORES)

    @functools.partial(
        pl.kernel,
        mesh=mesh,
        compiler_params=pltpu.CompilerParams(use_tc_tiling_on_sc=False),
        out_type=jax.ShapeDtypeStruct((batch, seq_len, dim), jnp.float32),
        scratch_types=(
            [pltpu.VMEM((chunk,), jnp.int32) for _ in range(NBUF)]
            + [pltpu.VMEM((chunk, dim), jnp.float32) for _ in range(NBUF)]
            + [pltpu.VMEM((seq_len, dim), jnp.float32)]
            + [pltpu.SemaphoreType.DMA for _ in range(2 * NBUF)]
        ),
    )
    def sc_kernel(idx_hbm, tok_hbm, pos_hbm, out_hbm, *scratch):
        idx_v = scratch[:NBUF]
        rows_v = scratch[NBUF:2 * NBUF]
        pos_v = scratch[2 * NBUF]
        sem_g = scratch[2 * NBUF + 1:2 * NBUF + 1 + NBUF]
        sem_s = scratch[2 * NBUF + 1 + NBUF:]

        wid = lax.axis_index("s") * NUM_CORES + lax.axis_index("c")
        tok_base = wid * per_w
        row_base = wid * rows_per_w
        pltpu.sync_copy(pos_hbm, pos_v)

        def issue_gather(c, b):
            off = tok_base + c * chunk
            pltpu.sync_copy(idx_hbm.at[pl.ds(off, chunk)], idx_v[b])
            pltpu.async_copy(tok_hbm.at[idx_v[b]], rows_v[b], sem_g[b])

        def wait_gather(b):
            pltpu.make_async_copy(tok_hbm.at[idx_v[b]], rows_v[b], sem_g[b]).wait()

        def issue_scatter(c, b):
            row0 = row_base + c * CHUNK_BATCH_ROWS
            for r in range(CHUNK_BATCH_ROWS):
                pltpu.async_copy(
                    rows_v[b].at[pl.ds(r * seq_len, seq_len)],
                    out_hbm.at[row0 + r], sem_s[b])

        def wait_scatter(b):
            for r in range(CHUNK_BATCH_ROWS):
                pltpu.make_async_copy(
                    rows_v[b].at[pl.ds(r * seq_len, seq_len)],
                    out_hbm.at[0], sem_s[b]).wait()

        def add_positions(rref):
            def s_body(s):
                p0 = pos_v[s, pl.ds(0, 16)]
                p1 = pos_v[s, pl.ds(16, 16)]
                for r in range(CHUNK_BATCH_ROWS):
                    j = r * seq_len + s
                    rref[j, pl.ds(0, 16)] = rref[j, pl.ds(0, 16)] + p0
                    rref[j, pl.ds(16, 16)] = rref[j, pl.ds(16, 16)] + p1

            plsc.parallel_loop(0, seq_len, 1, unroll=2)(s_body)

        for k in range(NBUF - 1):
            issue_gather(k, k)

        def outer(i, _):
            for b in range(NBUF):
                c = i * NBUF + b
                g = c + NBUF - 1
                gb = (b + NBUF - 1) % NBUF

                @pl.when(g < n_chunks)
                def _issue():
                    @pl.when(c >= 1)
                    def _drain():
                        wait_scatter(gb)

                    issue_gather(g, gb)

                wait_gather(b)
                add_positions(rows_v[b])
                issue_scatter(c, b)
            return ()

        lax.fori_loop(0, n_chunks // NBUF, outer, ())

        for b in range(NBUF):
            wait_scatter(b)

    return sc_kernel(flat_idx, token_table, position_table)


def kernel(inputs, token_table, position_table):
    b, s = inputs.shape
    dim = token_table.shape[1]
    flat_idx = inputs.reshape(b * s).astype(jnp.int32)
    return _pos_embed_sc(flat_idx, token_table, position_table, b, s, dim)


# speedup vs baseline: 5.4552x; 1.0639x over previous
import functools

import jax
import jax.numpy as jnp
from jax import lax
from jax.experimental import pallas as pl
from jax.experimental.pallas import tpu as pltpu
from jax.experimental.pallas import tpu_sc as plsc

NUM_CORES = 2
NUM_SUBCORES = 16
NUM_WORKERS = NUM_CORES * NUM_SUBCORES

S_CHUNK = 2
NBUF = 4
LANE = 16
BBLK = 128


@functools.partial(jax.jit, static_argnums=(3, 4, 5))
def _pos_embed_sc(perm_idx, token_table, position_table, batch, seq_len, dim):
    n = perm_idx.shape[0]
    per_w = n // NUM_WORKERS
    chunk = S_CHUNK * BBLK
    n_chunks = seq_len // S_CHUNK
    n_tc = dim // 8
    n_tb = batch // BBLK
    tsz = S_CHUNK * n_tc * 1024
    assert n_chunks % NBUF == 0 and per_w == n_chunks * chunk

    mesh = plsc.VectorSubcoreMesh(
        core_axis_name="c", subcore_axis_name="s",
        num_cores=NUM_CORES, num_subcores=NUM_SUBCORES)

    @functools.partial(
        pl.kernel,
        mesh=mesh,
        compiler_params=pltpu.CompilerParams(
            use_tc_tiling_on_sc=False, needs_layout_passes=False),
        out_type=jax.ShapeDtypeStruct((seq_len, n_tc, n_tb, 1024), jnp.float32),
        scratch_types=(
            [pltpu.VMEM((chunk,), jnp.int32) for _ in range(NBUF)]
            + [pltpu.VMEM((chunk, dim), jnp.float32) for _ in range(NBUF)]
            + [pltpu.VMEM((tsz,), jnp.float32) for _ in range(NBUF)]
            + [pltpu.VMEM((seq_len, dim), jnp.float32)]
            + [pltpu.SemaphoreType.DMA for _ in range(2 * NBUF)]
        ),
    )
    def sc_kernel(idx_hbm, tok_hbm, pos_hbm, out_hbm, *scratch):
        idx_v = scratch[:NBUF]
        gbuf = scratch[NBUF:2 * NBUF]
        tbuf = scratch[2 * NBUF:3 * NBUF]
        pos_v = scratch[3 * NBUF]
        sem_g = scratch[3 * NBUF + 1:3 * NBUF + 1 + NBUF]
        sem_s = scratch[3 * NBUF + 1 + NBUF:]

        wid = lax.axis_index("s") * NUM_CORES + lax.axis_index("c")
        tok_base = wid * per_w

        pltpu.sync_copy(pos_hbm, pos_v)
        ci = lax.iota(jnp.int32, LANE)
        pat = (ci >> 3) * 1024 + (ci & 7) * BBLK

        def issue_gather(c, b):
            off = tok_base + c * chunk
            pltpu.sync_copy(idx_hbm.at[pl.ds(off, chunk)], idx_v[b])
            pltpu.async_copy(tok_hbm.at[idx_v[b]], gbuf[b], sem_g[b])

        def wait_gather(b):
            pltpu.make_async_copy(tok_hbm.at[idx_v[b]], gbuf[b], sem_g[b]).wait()

        def issue_scatter(c, b):
            s0 = c * S_CHUNK
            for sl in range(S_CHUNK):
                for tc in range(n_tc):
                    pltpu.async_copy(
                        tbuf[b].at[pl.ds((sl * n_tc + tc) * 1024, 1024)],
                        out_hbm.at[s0 + sl, tc, wid], sem_s[b])

        def wait_scatter(b):
            for k in range(S_CHUNK * n_tc):
                pltpu.make_async_copy(
                    tbuf[b].at[pl.ds(k * 1024, 1024)],
                    out_hbm.at[0, 0, 0], sem_s[b]).wait()

        def transpose_add(c, b):
            s0 = c * S_CHUNK
            for sl in range(S_CHUNK):
                p0 = pos_v[s0 + sl, pl.ds(0, LANE)]
                p1 = pos_v[s0 + sl, pl.ds(LANE, LANE)]

                def bl_body(bl):
                    j = sl * BBLK + bl
                    base = pat + (sl * n_tc * 1024 + bl)
                    v0 = gbuf[b][j, pl.ds(0, LANE)] + p0
                    v1 = gbuf[b][j, pl.ds(LANE, LANE)] + p1
                    plsc.store_scatter(tbuf[b], [base], v0)
                    plsc.store_scatter(tbuf[b], [base + 2048], v1)

                plsc.parallel_loop(0, BBLK, 1, unroll=4)(bl_body)

        for k in range(NBUF - 1):
            issue_gather(k, k)

        def outer(i, _):
            for b in range(NBUF):
                c = i * NBUF + b
                g = c + NBUF - 1

                @pl.when(g < n_chunks)
                def _issue():
                    issue_gather(g, (b + NBUF - 1) % NBUF)

                wait_gather(b)

                @pl.when(c >= NBUF)
                def _drain():
                    wait_scatter(b)

                transpose_add(c, b)
                issue_scatter(c, b)
            return ()

        lax.fori_loop(0, n_chunks // NBUF, outer, ())

        for b in range(NBUF):
            wait_scatter(b)

    return sc_kernel(perm_idx, token_table, position_table)


def kernel(inputs, token_table, position_table):
    b, s = inputs.shape
    dim = token_table.shape[1]
    rows_per_w = b // NUM_WORKERS
    perm_idx = (inputs.astype(jnp.int32)
                .reshape(NUM_WORKERS, rows_per_w, s)
                .transpose(0, 2, 1)
                .reshape(b * s))
    raw = _pos_embed_sc(perm_idx, token_table, position_table, b, s, dim)
    return (raw.reshape(s, dim // 8, b // BBLK, 8, BBLK)
            .transpose(2, 4, 0, 1, 3)
            .reshape(b, s, dim))


# speedup vs baseline: 15.2084x; 2.7879x over previous
import functools

import jax
import jax.numpy as jnp
from jax import lax
from jax.experimental import pallas as pl
from jax.experimental.pallas import tpu as pltpu
from jax.experimental.pallas import tpu_sc as plsc

NUM_CORES = 2
NUM_SUBCORES = 16
NUM_WORKERS = NUM_CORES * NUM_SUBCORES

S_CHUNK = 2
NBUF = 4
LANE = 16
BBLK = 128
TPAD = BBLK + 1


@functools.partial(jax.jit, static_argnums=(3, 4, 5))
def _pos_embed_sc(perm_idx, token_table, position_table, batch, seq_len, dim):
    n = perm_idx.shape[0]
    per_w = n // NUM_WORKERS
    chunk = S_CHUNK * BBLK
    n_chunks = seq_len // S_CHUNK
    n_tc = dim // 8
    n_tb = batch // BBLK
    trows = S_CHUNK * n_tc * 8
    assert n_chunks % NBUF == 0 and per_w == n_chunks * chunk

    mesh = plsc.VectorSubcoreMesh(
        core_axis_name="c", subcore_axis_name="s",
        num_cores=NUM_CORES, num_subcores=NUM_SUBCORES)

    @functools.partial(
        pl.kernel,
        mesh=mesh,
        compiler_params=pltpu.CompilerParams(
            use_tc_tiling_on_sc=False, needs_layout_passes=False),
        out_type=jax.ShapeDtypeStruct(
            (seq_len, n_tc, n_tb, 8, BBLK), jnp.float32),
        scratch_types=(
            [pltpu.VMEM((chunk,), jnp.int32) for _ in range(NBUF)]
            + [pltpu.VMEM((chunk, dim), jnp.float32) for _ in range(NBUF)]
            + [pltpu.VMEM((trows, TPAD), jnp.float32) for _ in range(NBUF)]
            + [pltpu.VMEM((seq_len, dim), jnp.float32)]
            + [pltpu.SemaphoreType.DMA for _ in range(2 * NBUF)]
        ),
    )
    def sc_kernel(idx_hbm, tok_hbm, pos_hbm, out_hbm, *scratch):
        idx_v = scratch[:NBUF]
        gbuf = scratch[NBUF:2 * NBUF]
        tbuf = scratch[2 * NBUF:3 * NBUF]
        pos_v = scratch[3 * NBUF]
        sem_g = scratch[3 * NBUF + 1:3 * NBUF + 1 + NBUF]
        sem_s = scratch[3 * NBUF + 1 + NBUF:]

        wid = lax.axis_index("s") * NUM_CORES + lax.axis_index("c")
        tok_base = wid * per_w

        pltpu.sync_copy(pos_hbm, pos_v)
        ci = lax.iota(jnp.int32, LANE)
        rowp = (ci >> 3) * 8 + (ci & 7)

        def issue_gather(c, b):
            off = tok_base + c * chunk
            pltpu.sync_copy(idx_hbm.at[pl.ds(off, chunk)], idx_v[b])
            pltpu.async_copy(tok_hbm.at[idx_v[b]], gbuf[b], sem_g[b])

        def wait_gather(b):
            pltpu.make_async_copy(tok_hbm.at[idx_v[b]], gbuf[b], sem_g[b]).wait()

        def issue_scatter(c, b):
            s0 = c * S_CHUNK
            for sl in range(S_CHUNK):
                for tc in range(n_tc):
                    pltpu.async_copy(
                        tbuf[b].at[pl.ds((sl * n_tc + tc) * 8, 8), pl.ds(0, BBLK)],
                        out_hbm.at[s0 + sl, tc, wid], sem_s[b])

        def wait_scatter(b):
            for _ in range(S_CHUNK * n_tc):
                pltpu.make_async_copy(
                    tbuf[b].at[pl.ds(0, 8), pl.ds(0, BBLK)],
                    out_hbm.at[0, 0, 0], sem_s[b]).wait()

        def transpose_add(c, b):
            s0 = c * S_CHUNK
            for sl in range(S_CHUNK):
                p0 = pos_v[s0 + sl, pl.ds(0, LANE)]
                p1 = pos_v[s0 + sl, pl.ds(LANE, LANE)]
                row0 = rowp + sl * (n_tc * 8)
                row1 = row0 + 2 * 8

                def bl_body(bl):
                    j = sl * BBLK + bl
                    col = jnp.full((LANE,), bl, jnp.int32)
                    v0 = gbuf[b][j, pl.ds(0, LANE)] + p0
                    v1 = gbuf[b][j, pl.ds(LANE, LANE)] + p1
                    plsc.store_scatter(tbuf[b], [row0, col], v0)
                    plsc.store_scatter(tbuf[b], [row1, col], v1)

                plsc.parallel_loop(0, BBLK, 1, unroll=4)(bl_body)

        for k in range(NBUF - 1):
            issue_gather(k, k)

        def outer(i, _):
            for b in range(NBUF):
                c = i * NBUF + b
                g = c + NBUF - 1

                @pl.when(g < n_chunks)
                def _issue():
                    issue_gather(g, (b + NBUF - 1) % NBUF)

                wait_gather(b)

                @pl.when(c >= NBUF)
                def _drain():
                    wait_scatter(b)

                transpose_add(c, b)
                issue_scatter(c, b)
            return ()

        lax.fori_loop(0, n_chunks // NBUF, outer, ())

        for b in range(NBUF):
            wait_scatter(b)

    return sc_kernel(perm_idx, token_table, position_table)


def kernel(inputs, token_table, position_table):
    b, s = inputs.shape
    dim = token_table.shape[1]
    rows_per_w = b // NUM_WORKERS
    perm_idx = (inputs.astype(jnp.int32)
                .reshape(NUM_WORKERS, rows_per_w, s)
                .transpose(0, 2, 1)
                .reshape(b * s))
    raw = _pos_embed_sc(perm_idx, token_table, position_table, b, s, dim)
    return raw.transpose(2, 4, 0, 1, 3).reshape(b, s, dim)
